# baseline (speedup 1.0000x reference)
import functools

import jax
import jax.numpy as jnp
from jax import lax
from jax.experimental import pallas as pl
from jax.experimental.pallas import tpu as pltpu
from jax.experimental.pallas import tpu_sc as plsc

_NUM_CORES = 2
_NUM_SUBCORES = 16
_NW = _NUM_CORES * _NUM_SUBCORES
_LANES = 16


def _sc_gather_rel(t_pad, h, w):
    c = t_pad.shape[1]
    hw = h * w
    rows_per_tile = hw // _NW
    mesh = plsc.VectorSubcoreMesh(core_axis_name="c", subcore_axis_name="s")

    @functools.partial(
        pl.kernel,
        out_type=jax.ShapeDtypeStruct((hw, c), jnp.float32),
        mesh=mesh,
        scratch_types=[
            pltpu.VMEM((rows_per_tile,), jnp.int32),
            pltpu.VMEM((rows_per_tile, c), jnp.float32),
            pltpu.SemaphoreType.DMA,
        ],
    )
    def rel_kernel(t_hbm, rel_hbm, idx_v, rows_v, sem):
        wid = lax.axis_index("s") * _NUM_CORES + lax.axis_index("c")
        base = wid * rows_per_tile
        lane = lax.iota(jnp.int32, _LANES)
        for k in range(rows_per_tile // _LANES):
            pos = base + k * _LANES + lane
            hh = jnp.right_shift(pos, w.bit_length() - 1)
            ww = jnp.bitwise_and(pos, w - 1)
            idx_v[pl.ds(k * _LANES, _LANES)] = ww - hh + (w - 1)
        pltpu.async_copy(t_hbm.at[idx_v], rows_v, sem).wait()
        pltpu.sync_copy(rows_v, rel_hbm.at[pl.ds(base, rows_per_tile)])

    return rel_kernel(t_pad)


_NBUF = 4


def _tc_add(xt, rel):
    b, h, w, c = xt.shape

    def body(x_hbm, rel_v, o_hbm, xbuf, obuf, in_sems, out_sems):
        def in_copy(i, slot):
            return pltpu.make_async_copy(
                x_hbm.at[i], xbuf.at[slot], in_sems.at[slot])

        def out_copy(i, slot):
            return pltpu.make_async_copy(
                obuf.at[slot], o_hbm.at[i], out_sems.at[slot])

        for i in range(_NBUF):
            in_copy(i, i).start()
        for i in range(b):
            slot = i % _NBUF
            in_copy(i, slot).wait()
            if i >= _NBUF:
                out_copy(i - _NBUF, slot).wait()
            obuf[slot] = xbuf[slot] + rel_v[...]
            out_copy(i, slot).start()
            if i + _NBUF < b:
                in_copy(i + _NBUF, slot).start()
        for i in range(b - _NBUF, b):
            out_copy(i, i % _NBUF).wait()

    return pl.pallas_call(
        body,
        grid=(1,),
        in_specs=[
            pl.BlockSpec(memory_space=pl.ANY),
            pl.BlockSpec((h, w, c), lambda _: (0, 0, 0)),
        ],
        out_specs=pl.BlockSpec(memory_space=pl.ANY),
        out_shape=jax.ShapeDtypeStruct((b, h, w, c), jnp.float32),
        scratch_shapes=[
            pltpu.VMEM((_NBUF, h, w, c), jnp.float32),
            pltpu.VMEM((_NBUF, h, w, c), jnp.float32),
            pltpu.SemaphoreType.DMA((_NBUF,)),
            pltpu.SemaphoreType.DMA((_NBUF,)),
        ],
    )(xt, rel)


def kernel(x, rel_emb_x, rel_emb_y):
    b, c, h, w = x.shape
    t = jnp.concatenate([rel_emb_x, rel_emb_y], axis=1)
    t_pad = jnp.pad(t, ((0, 1), (0, 0)))
    rel = _sc_gather_rel(t_pad, h, w).reshape(h, w, c)
    xt = jnp.transpose(x, (0, 2, 3, 1))
    out = _tc_add(xt, rel)
    return jnp.transpose(out, (0, 3, 1, 2))


# speedup vs baseline: 1.0045x; 1.0045x over previous
import functools

import jax
import jax.numpy as jnp
from jax import lax
from jax.experimental import pallas as pl
from jax.experimental.pallas import tpu as pltpu
from jax.experimental.pallas import tpu_sc as plsc

_NUM_CORES = 2
_NUM_SUBCORES = 16
_NW = _NUM_CORES * _NUM_SUBCORES
_LANES = 16


def _sc_gather_rel(t_pad, h, w):
    c = t_pad.shape[1]
    hw = h * w
    rows_per_tile = hw // _NW
    mesh = plsc.VectorSubcoreMesh(core_axis_name="c", subcore_axis_name="s")

    @functools.partial(
        pl.kernel,
        out_type=jax.ShapeDtypeStruct((hw, c), jnp.float32),
        mesh=mesh,
        scratch_types=[
            pltpu.VMEM((rows_per_tile,), jnp.int32),
            pltpu.VMEM((rows_per_tile, c), jnp.float32),
            pltpu.SemaphoreType.DMA,
        ],
    )
    def rel_kernel(t_hbm, rel_hbm, idx_v, rows_v, sem):
        wid = lax.axis_index("s") * _NUM_CORES + lax.axis_index("c")
        base = wid * rows_per_tile
        lane = lax.iota(jnp.int32, _LANES)
        for k in range(rows_per_tile // _LANES):
            pos = base + k * _LANES + lane
            hh = jnp.right_shift(pos, w.bit_length() - 1)
            ww = jnp.bitwise_and(pos, w - 1)
            idx_v[pl.ds(k * _LANES, _LANES)] = ww - hh + (w - 1)
        pltpu.async_copy(t_hbm.at[idx_v], rows_v, sem).wait()
        pltpu.sync_copy(rows_v, rel_hbm.at[pl.ds(base, rows_per_tile)])

    return rel_kernel(t_pad)


_NBUF = 4


def _tc_add_block_body(x_ref, rel_ref, o_ref):
    o_ref[...] = x_ref[...] + rel_ref[...]


def _tc_add(xt, rel):
    b, h, w, c = xt.shape
    tile_b = 2
    grid = (b // tile_b,)
    return pl.pallas_call(
        _tc_add_block_body,
        grid=grid,
        in_specs=[
            pl.BlockSpec((tile_b, h, w, c), lambda bi: (bi, 0, 0, 0)),
            pl.BlockSpec((h, w, c), lambda bi: (0, 0, 0)),
        ],
        out_specs=pl.BlockSpec((tile_b, h, w, c), lambda bi: (bi, 0, 0, 0)),
        out_shape=jax.ShapeDtypeStruct((b, h, w, c), jnp.float32),
    )(xt, rel)

    def body(x_hbm, rel_v, o_hbm, xbuf, obuf, in_sems, out_sems):
        def in_copy(i, slot):
            return pltpu.make_async_copy(
                x_hbm.at[i], xbuf.at[slot], in_sems.at[slot])

        def out_copy(i, slot):
            return pltpu.make_async_copy(
                obuf.at[slot], o_hbm.at[i], out_sems.at[slot])

        for i in range(_NBUF):
            in_copy(i, i).start()
        for i in range(b):
            slot = i % _NBUF
            in_copy(i, slot).wait()
            if i >= _NBUF:
                out_copy(i - _NBUF, slot).wait()
            obuf[slot] = xbuf[slot] + rel_v[...]
            out_copy(i, slot).start()
            if i + _NBUF < b:
                in_copy(i + _NBUF, slot).start()
        for i in range(b - _NBUF, b):
            out_copy(i, i % _NBUF).wait()

    return pl.pallas_call(
        body,
        grid=(1,),
        in_specs=[
            pl.BlockSpec(memory_space=pl.ANY),
            pl.BlockSpec((h, w, c), lambda _: (0, 0, 0)),
        ],
        out_specs=pl.BlockSpec(memory_space=pl.ANY),
        out_shape=jax.ShapeDtypeStruct((b, h, w, c), jnp.float32),
        scratch_shapes=[
            pltpu.VMEM((_NBUF, h, w, c), jnp.float32),
            pltpu.VMEM((_NBUF, h, w, c), jnp.float32),
            pltpu.SemaphoreType.DMA((_NBUF,)),
            pltpu.SemaphoreType.DMA((_NBUF,)),
        ],
    )(xt, rel)


def kernel(x, rel_emb_x, rel_emb_y):
    b, c, h, w = x.shape
    t = jnp.concatenate([rel_emb_x, rel_emb_y], axis=1)
    t_pad = jnp.pad(t, ((0, 1), (0, 0)))
    rel = _sc_gather_rel(t_pad, h, w).reshape(h, w, c)
    xt = jnp.transpose(x, (0, 2, 3, 1))
    out = _tc_add(xt, rel)
    return jnp.transpose(out, (0, 3, 1, 2))
